# baseline (device time: 448693 ns/iter reference)
import jax
import jax.numpy as jnp
from jax import lax
from jax.experimental import pallas as pl
from jax.experimental.pallas import tpu as pltpu

N_DEV = 4


def _gather_body(x_ref, w_ref, xg_ref, wg_ref, copy_sems, sems):
    my = lax.axis_index("i")
    left = (my - 1) % N_DEV
    right = (my + 1) % N_DEV
    m, kl = x_ref.shape
    mh = m // 2
    kh = kl // 2

    barrier = pltpu.get_barrier_semaphore()
    for nbr in (left, right):
        pl.semaphore_signal(barrier, inc=1, device_id=(nbr,),
                            device_id_type=pl.DeviceIdType.MESH)
    pl.semaphore_wait(barrier, 2)

    cx = pltpu.make_async_copy(x_ref, xg_ref.at[my], copy_sems.at[0])
    cw = pltpu.make_async_copy(w_ref, wg_ref.at[my], copy_sems.at[1])
    cx.start()
    cw.start()

    def rdma(src, dst, s, r, dev):
        return pltpu.make_async_remote_copy(
            src_ref=src, dst_ref=dst, send_sem=sems.at[s], recv_sem=sems.at[r],
            device_id=(dev,), device_id_type=pl.DeviceIdType.MESH)

    p1 = [
        rdma(x_ref, xg_ref.at[my], 0, 1, right),
        rdma(w_ref, wg_ref.at[my], 2, 3, right),
        rdma(x_ref, xg_ref.at[my], 4, 5, left),
        rdma(w_ref, wg_ref.at[my], 6, 7, left),
    ]
    for d in p1:
        d.start()
    for d in p1:
        d.wait()
    cx.wait()
    cw.wait()

    p2 = [
        rdma(xg_ref.at[right, :mh], xg_ref.at[right, :mh], 8, 9, left),
        rdma(wg_ref.at[right, :kh], wg_ref.at[right, :kh], 10, 11, left),
        rdma(xg_ref.at[left, mh:], xg_ref.at[left, mh:], 12, 13, right),
        rdma(wg_ref.at[left, kh:], wg_ref.at[left, kh:], 14, 15, right),
    ]
    for d in p2:
        d.start()
    for d in p2:
        d.wait()


def _gather(x, w_mat):
    m, kl = x.shape
    _, n = w_mat.shape
    return pl.pallas_call(
        _gather_body,
        out_shape=[
            jax.ShapeDtypeStruct((N_DEV, m, kl), x.dtype),
            jax.ShapeDtypeStruct((N_DEV, kl, n), w_mat.dtype),
        ],
        in_specs=[pl.BlockSpec(memory_space=pl.ANY),
                  pl.BlockSpec(memory_space=pl.ANY)],
        out_specs=[pl.BlockSpec(memory_space=pl.ANY),
                   pl.BlockSpec(memory_space=pl.ANY)],
        scratch_shapes=[
            pltpu.SemaphoreType.DMA((2,)),
            pltpu.SemaphoreType.DMA((16,)),
        ],
        compiler_params=pltpu.CompilerParams(collective_id=0),
    )(x, w_mat)


def _gemm_body(xg_ref, wg_ref, sx_ref, sw_ref, out_ref):
    acc = jnp.dot(xg_ref[0], wg_ref[0], preferred_element_type=jnp.float32)
    for j in range(1, N_DEV):
        acc += jnp.dot(xg_ref[j], wg_ref[j],
                       preferred_element_type=jnp.float32)
    s = sx_ref[0] * sw_ref[0]
    out_ref[...] = jnp.maximum(acc * s, 0.0)


def _gemm(xg, wg, scale_x, scale_w):
    _, m, kl = xg.shape
    _, _, n = wg.shape
    nt = 512
    return pl.pallas_call(
        _gemm_body,
        grid=(n // nt,),
        out_shape=jax.ShapeDtypeStruct((m, n), jnp.float32),
        in_specs=[
            pl.BlockSpec((N_DEV, m, kl), lambda i: (0, 0, 0)),
            pl.BlockSpec((N_DEV, kl, nt), lambda i: (0, 0, i)),
            pl.BlockSpec(memory_space=pltpu.MemorySpace.SMEM),
            pl.BlockSpec(memory_space=pltpu.MemorySpace.SMEM),
        ],
        out_specs=pl.BlockSpec((m, nt), lambda i: (0, i)),
    )(xg, wg, scale_x, scale_w)


NT = 512
NTILES = 8192 // NT


def _fused_body(x_ref, w_ref, sx_ref, sw_ref, out_ref, wg_ref,
                xg, wt, ob, ao, csem, sems, wt_sems, ob_sems, ao_sems):
    my = lax.axis_index("i")
    left = (my - 1) % N_DEV
    right = (my + 1) % N_DEV
    diag = (my + 2) % N_DEV
    m, kl = x_ref.shape
    mh = m // 2
    kh = kl // 2

    barrier = pltpu.get_barrier_semaphore()
    for nbr in (left, right):
        pl.semaphore_signal(barrier, inc=1, device_id=(nbr,),
                            device_id_type=pl.DeviceIdType.MESH)
    pl.semaphore_wait(barrier, 2)

    def rdma(src, dst, s_i, r_i, dev):
        return pltpu.make_async_remote_copy(
            src_ref=src, dst_ref=dst,
            send_sem=sems.at[s_i], recv_sem=sems.at[r_i],
            device_id=(dev,), device_id_type=pl.DeviceIdType.MESH)

    cx = pltpu.make_async_copy(x_ref, xg.at[my], csem)
    cx.start()
    p1 = [
        rdma(x_ref, xg.at[my], 0, 1, right),
        rdma(x_ref, xg.at[my], 2, 3, left),
        rdma(w_ref, wg_ref.at[my], 4, 5, right),
        rdma(w_ref, wg_ref.at[my], 6, 7, left),
    ]
    for d in p1:
        d.start()

    s = sx_ref[0] * sw_ref[0]

    def run_pass(chunks, w_loader, accumulate, epilogue):
        n_s = len(chunks)

        def issue(nt):
            b = lax.rem(nt, 2)
            for si in range(n_s):
                w_loader(si, nt, b).start()
            if accumulate:
                pltpu.make_async_copy(
                    out_ref.at[:, pl.ds(nt * NT, NT)], ob.at[b],
                    ob_sems.at[b]).start()

        issue(0)

        def body(nt, carry):
            b = lax.rem(nt, 2)

            @pl.when(nt < NTILES - 1)
            def _():
                issue(nt + 1)

            for si in range(n_s):
                w_loader(si, nt, b).wait()
            acc = jnp.dot(xg[chunks[0]], wt[0, b],
                          preferred_element_type=jnp.float32)
            for si in range(1, n_s):
                acc = acc + jnp.dot(xg[chunks[si]], wt[si, b],
                                    preferred_element_type=jnp.float32)
            if accumulate:
                pltpu.make_async_copy(
                    out_ref.at[:, pl.ds(nt * NT, NT)], ob.at[b],
                    ob_sems.at[b]).wait()
                acc = acc + ob[b]
            if epilogue:
                acc = jnp.maximum(acc * s, 0.0)

            @pl.when(nt >= 2)
            def _():
                pltpu.make_async_copy(
                    ao.at[b], out_ref.at[:, pl.ds((nt - 2) * NT, NT)],
                    ao_sems.at[b]).wait()

            ao[b] = acc
            pltpu.make_async_copy(
                ao.at[b], out_ref.at[:, pl.ds(nt * NT, NT)],
                ao_sems.at[b]).start()
            return carry

        lax.fori_loop(0, NTILES, body, 0)
        pltpu.make_async_copy(
            ao.at[0], out_ref.at[:, pl.ds((NTILES - 2) * NT, NT)],
            ao_sems.at[0]).wait()
        pltpu.make_async_copy(
            ao.at[1], out_ref.at[:, pl.ds((NTILES - 1) * NT, NT)],
            ao_sems.at[1]).wait()

    def w_local(si, nt, b):
        return pltpu.make_async_copy(
            w_ref.at[:, pl.ds(nt * NT, NT)], wt.at[0, b], wt_sems.at[b])

    def w_chunk(c, si, nt, b):
        return pltpu.make_async_copy(
            wg_ref.at[c, :, pl.ds(nt * NT, NT)], wt.at[si, b],
            wt_sems.at[si * 2 + b])

    cx.wait()
    run_pass([my], w_local, accumulate=False, epilogue=False)

    for d in p1:
        d.wait()

    p2 = [
        rdma(xg.at[right, pl.ds(0, mh)], xg.at[right, pl.ds(0, mh)],
             8, 9, left),
        rdma(wg_ref.at[right, pl.ds(0, kh)], wg_ref.at[right, pl.ds(0, kh)],
             10, 11, left),
        rdma(xg.at[left, pl.ds(mh, mh)], xg.at[left, pl.ds(mh, mh)],
             12, 13, right),
        rdma(wg_ref.at[left, pl.ds(kh, kh)], wg_ref.at[left, pl.ds(kh, kh)],
             14, 15, right),
    ]
    for d in p2:
        d.start()

    run_pass([left, right],
             lambda si, nt, b: w_chunk(left if si == 0 else right, si, nt, b),
             accumulate=True, epilogue=False)

    for d in p2:
        d.wait()

    run_pass([diag], lambda si, nt, b: w_chunk(diag, si, nt, b),
             accumulate=True, epilogue=True)


def _fused(x8, w8, scale_x, scale_w):
    m, kl = x8.shape
    _, n = w8.shape
    out, _wg = pl.pallas_call(
        _fused_body,
        out_shape=[
            jax.ShapeDtypeStruct((m, n), jnp.float32),
            jax.ShapeDtypeStruct((N_DEV, kl, n), x8.dtype),
        ],
        in_specs=[
            pl.BlockSpec(memory_space=pl.ANY),
            pl.BlockSpec(memory_space=pl.ANY),
            pl.BlockSpec(memory_space=pltpu.MemorySpace.SMEM),
            pl.BlockSpec(memory_space=pltpu.MemorySpace.SMEM),
        ],
        out_specs=[pl.BlockSpec(memory_space=pl.ANY),
                   pl.BlockSpec(memory_space=pl.ANY)],
        scratch_shapes=[
            pltpu.VMEM((N_DEV, m, kl), x8.dtype),
            pltpu.VMEM((2, 2, kl, NT), x8.dtype),
            pltpu.VMEM((2, m, NT), jnp.float32),
            pltpu.VMEM((2, m, NT), jnp.float32),
            pltpu.SemaphoreType.DMA,
            pltpu.SemaphoreType.DMA((16,)),
            pltpu.SemaphoreType.DMA((4,)),
            pltpu.SemaphoreType.DMA((2,)),
            pltpu.SemaphoreType.DMA((2,)),
        ],
        compiler_params=pltpu.CompilerParams(
            collective_id=0, vmem_limit_bytes=64 * 1024 * 1024),
    )(x8, w8, scale_x, scale_w)
    return out


def kernel(x, w_mat, scale_x, scale_w):
    x8 = x.astype(jnp.float8_e4m3fn)
    w8 = w_mat.astype(jnp.float8_e4m3fn)
    return _fused(x8, w8, scale_x, scale_w)
